# baseline (speedup 1.0000x reference)
import jax
import jax.numpy as jnp
from jax import lax
from jax.experimental import pallas as pl
from jax.experimental.pallas import tpu as pltpu
from jax.experimental.pallas import tpu_sc as plsc

NC, NS = 2, 16
NW = NC * NS
B, H, D, NL = 4096, 200, 64, 128
BPW = B // NW
C0, C1 = 104, 96


def _sc_pool_body(idx_hbm, emb_hbm, out_hbm, idx_v, rows_v, acc_v, sem):
    c = lax.axis_index("c")
    s = lax.axis_index("s")
    wid = s * NC + c
    base = wid * BPW
    pltpu.sync_copy(idx_hbm.at[pl.ds(base, BPW)], idx_v)

    def row_body(r, carry):
        cp0 = pltpu.async_copy(
            emb_hbm.at[idx_v.at[r, pl.ds(0, C0)]], rows_v.at[pl.ds(0, C0)], sem)
        cp1 = pltpu.async_copy(
            emb_hbm.at[idx_v.at[r, pl.ds(C0, C1)]], rows_v.at[pl.ds(C0, C1)], sem)
        cp0.wait()
        cp1.wait()

        def jbody(j, accs):
            return tuple(accs[k] + rows_v[j, pl.ds(16 * k, 16)] for k in range(4))

        accs = lax.fori_loop(
            0, H, jbody,
            tuple(jnp.zeros((16,), jnp.float32) for _ in range(4)),
            unroll=4)
        for k in range(4):
            acc_v[r, pl.ds(16 * k, 16)] = accs[k]
        return carry

    lax.fori_loop(0, BPW, row_body, 0)
    pltpu.sync_copy(acc_v, out_hbm.at[pl.ds(base, BPW)])


def _sc_pool(input_ids, emb):
    mesh = plsc.VectorSubcoreMesh(
        core_axis_name="c", subcore_axis_name="s",
        num_cores=NC, num_subcores=NS)
    return pl.kernel(
        _sc_pool_body,
        out_type=jax.ShapeDtypeStruct((B, D), jnp.float32),
        mesh=mesh,
        scratch_types=[
            pltpu.VMEM((BPW, H), jnp.int32),
            pltpu.VMEM((H, D), jnp.float32),
            pltpu.VMEM((BPW, D), jnp.float32),
            pltpu.SemaphoreType.DMA,
        ],
        compiler_params=pltpu.CompilerParams(use_tc_tiling_on_sc=False),
    )(input_ids, emb)


def _tc_matmul_body(x_ref, wt_ref, b_ref, o_ref):
    x = x_ref[...] * (1.0 / H)
    o_ref[...] = jnp.dot(x, wt_ref[...],
                         preferred_element_type=jnp.float32) + b_ref[...]


def _tc_matmul(x, wt, b2):
    return pl.pallas_call(
        _tc_matmul_body,
        out_shape=jax.ShapeDtypeStruct((B, NL), jnp.float32),
    )(x, wt, b2)


def kernel(input, emb, W, b):
    pooled = _sc_pool(input, emb)
    return _tc_matmul(pooled, W.T, b.reshape(1, NL))


# speedup vs baseline: 1.6355x; 1.6355x over previous
import jax
import jax.numpy as jnp
from jax import lax
from jax.experimental import pallas as pl
from jax.experimental.pallas import tpu as pltpu
from jax.experimental.pallas import tpu_sc as plsc

NC, NS = 2, 16
NW = NC * NS
B, H, D, NL = 4096, 200, 64, 128
BPW = B // NW
C0, C1 = 104, 96


def _sc_pool_body(idx_hbm, emb_hbm, out_hbm, idx_v, rows_v, acc_v, sem):
    c = lax.axis_index("c")
    s = lax.axis_index("s")
    wid = s * NC + c
    base = wid * BPW
    pltpu.sync_copy(idx_hbm.at[pl.ds(base, BPW)], idx_v)

    def row_body(r, carry):
        cp0 = pltpu.async_copy(
            emb_hbm.at[idx_v.at[r, pl.ds(0, C0)]], rows_v.at[pl.ds(0, C0)], sem)
        cp1 = pltpu.async_copy(
            emb_hbm.at[idx_v.at[r, pl.ds(C0, C1)]], rows_v.at[pl.ds(C0, C1)], sem)
        cp0.wait()
        cp1.wait()

        def jbody(j, accs):
            return tuple(accs[k] + rows_v[j, pl.ds(16 * k, 16)] for k in range(4))

        accs = lax.fori_loop(
            0, H, jbody,
            tuple(jnp.zeros((16,), jnp.float32) for _ in range(4)),
            unroll=4)
        for k in range(4):
            acc_v[r, pl.ds(16 * k, 16)] = accs[k]
        return carry

    lax.fori_loop(0, BPW, row_body, 0)
    pltpu.sync_copy(acc_v, out_hbm.at[pl.ds(base, BPW)])


def _sc_pool(input_ids, emb):
    mesh = plsc.VectorSubcoreMesh(
        core_axis_name="c", subcore_axis_name="s",
        num_cores=NC, num_subcores=NS)
    return pl.kernel(
        _sc_pool_body,
        out_type=jax.ShapeDtypeStruct((B, D), jnp.float32),
        mesh=mesh,
        scratch_types=[
            pltpu.VMEM((BPW, H), jnp.int32),
            pltpu.VMEM((H, D), jnp.float32),
            pltpu.VMEM((BPW, D), jnp.float32),
            pltpu.SemaphoreType.DMA,
        ],
        compiler_params=pltpu.CompilerParams(use_tc_tiling_on_sc=False),
    )(input_ids, emb)


VB = 8192
RB = VB // 2
V = 1000000


def _tc_transpose_body(x_ref, o_ref):
    o_ref[:, 0:64] = x_ref[:, 0:RB].T
    o_ref[:, 64:128] = x_ref[:, RB:VB].T


NMAIN = (V // VB) * VB
NTAIL = V - NMAIN
HT = NTAIL // 2


def _tc_tail_body(x_ref, tbl_ref, o_ref):
    del tbl_ref
    o_ref[0:HT, 0:64] = x_ref[:, 0:HT].T
    o_ref[0:HT, 64:128] = x_ref[:, HT:NTAIL].T


def _tc_transpose(emt):
    nblk = NMAIN // VB
    main = pl.pallas_call(
        _tc_transpose_body,
        grid=(nblk,),
        in_specs=[pl.BlockSpec((64, VB), lambda i: (0, i))],
        out_specs=pl.BlockSpec((RB, 128), lambda i: (i, 0)),
        out_shape=jax.ShapeDtypeStruct((V // 2, 128), jnp.float32),
    )(emt)
    return pl.pallas_call(
        _tc_tail_body,
        grid=(1,),
        in_specs=[pl.BlockSpec((64, NTAIL), lambda i: (0, 0)),
                  pl.BlockSpec(memory_space=pl.ANY)],
        out_specs=pl.BlockSpec((RB, 128), lambda i: (NMAIN // VB, 0)),
        out_shape=jax.ShapeDtypeStruct((V // 2, 128), jnp.float32),
        input_output_aliases={1: 0},
    )(emt[:, NMAIN:], main)


def _tc_remap_body(i_ref, o_ref):
    v = i_ref[...]
    j = v & (VB - 1)
    main_row = (v - j) + 2 * (j & (RB - 1)) + (j >> 12)
    jt = v - NMAIN
    tail_row = NMAIN + 2 * (jt % HT) + jt // HT
    o_ref[...] = jnp.where(v >= NMAIN, tail_row, main_row)


def _tc_remap(input_ids):
    return pl.pallas_call(
        _tc_remap_body,
        out_shape=jax.ShapeDtypeStruct((B, H), jnp.int32),
    )(input_ids)


def _tc_matmul_body(x_ref, wt_ref, b_ref, o_ref):
    x = x_ref[...] * (1.0 / H)
    o_ref[...] = jnp.dot(x, wt_ref[...],
                         preferred_element_type=jnp.float32) + b_ref[...]


def _tc_matmul(x, wt, b2):
    return pl.pallas_call(
        _tc_matmul_body,
        out_shape=jax.ShapeDtypeStruct((B, NL), jnp.float32),
    )(x, wt, b2)


def kernel(input, emb, W, b):
    lin = _tc_transpose(emb.T).reshape(V, 64)
    pooled = _sc_pool(_tc_remap(input), lin)
    return _tc_matmul(pooled, W.T, b.reshape(1, NL))


# speedup vs baseline: 2.0335x; 1.2433x over previous
import jax
import jax.numpy as jnp
from jax import lax
from jax.experimental import pallas as pl
from jax.experimental.pallas import tpu as pltpu
from jax.experimental.pallas import tpu_sc as plsc

NC, NS = 2, 16
NW = NC * NS
B, H, D, NL = 4096, 200, 64, 128
BPW = B // NW
C0, C1 = 104, 96


def _sc_pool_body(idx_hbm, emb_hbm, out_hbm, idx_v, rows_a, rows_b, acc_v,
                  sem_a, sem_b):
    c = lax.axis_index("c")
    s = lax.axis_index("s")
    wid = s * NC + c
    base = wid * BPW
    pltpu.sync_copy(idx_hbm.at[pl.ds(base, BPW)], idx_v)

    def start(r, rows_ref, sem):
        pltpu.async_copy(
            emb_hbm.at[idx_v.at[r, pl.ds(0, C0)]], rows_ref.at[pl.ds(0, C0)], sem)
        pltpu.async_copy(
            emb_hbm.at[idx_v.at[r, pl.ds(C0, C1)]], rows_ref.at[pl.ds(C0, C1)], sem)

    def wait(r, rows_ref, sem):
        pltpu.make_async_copy(
            emb_hbm.at[idx_v.at[r, pl.ds(0, C0)]], rows_ref.at[pl.ds(0, C0)], sem).wait()
        pltpu.make_async_copy(
            emb_hbm.at[idx_v.at[r, pl.ds(C0, C1)]], rows_ref.at[pl.ds(C0, C1)], sem).wait()

    def reduce(rows_ref, r):
        def jbody(j, accs):
            return tuple(accs[k] + rows_ref[j, pl.ds(16 * k, 16)] for k in range(4))
        accs = lax.fori_loop(
            0, H, jbody,
            tuple(jnp.zeros((16,), jnp.float32) for _ in range(4)),
            unroll=8)
        for k in range(4):
            acc_v[r, pl.ds(16 * k, 16)] = accs[k]

    start(0, rows_a, sem_a)

    def body2(k, carry):
        r = 2 * k
        start(r + 1, rows_b, sem_b)
        wait(r, rows_a, sem_a)
        reduce(rows_a, r)

        @pl.when(k < BPW // 2 - 1)
        def _():
            start(r + 2, rows_a, sem_a)

        wait(r + 1, rows_b, sem_b)
        reduce(rows_b, r + 1)
        return carry

    lax.fori_loop(0, BPW // 2, body2, 0)
    pltpu.sync_copy(acc_v, out_hbm.at[pl.ds(base, BPW)])


def _sc_pool(input_ids, emb):
    mesh = plsc.VectorSubcoreMesh(
        core_axis_name="c", subcore_axis_name="s",
        num_cores=NC, num_subcores=NS)
    return pl.kernel(
        _sc_pool_body,
        out_type=jax.ShapeDtypeStruct((B, D), jnp.float32),
        mesh=mesh,
        scratch_types=[
            pltpu.VMEM((BPW, H), jnp.int32),
            pltpu.VMEM((H, D), jnp.float32),
            pltpu.VMEM((H, D), jnp.float32),
            pltpu.VMEM((BPW, D), jnp.float32),
            pltpu.SemaphoreType.DMA,
            pltpu.SemaphoreType.DMA,
        ],
        compiler_params=pltpu.CompilerParams(use_tc_tiling_on_sc=False),
    )(input_ids, emb)


VB = 8192
RB = VB // 2
V = 1000000


def _tc_transpose_body(x_ref, o_ref):
    o_ref[:, 0:64] = x_ref[:, 0:RB].T
    o_ref[:, 64:128] = x_ref[:, RB:VB].T


NMAIN = (V // VB) * VB
NTAIL = V - NMAIN
HT = NTAIL // 2


def _tc_tail_body(x_ref, tbl_ref, o_ref):
    del tbl_ref
    o_ref[0:HT, 0:64] = x_ref[:, 0:HT].T
    o_ref[0:HT, 64:128] = x_ref[:, HT:NTAIL].T


def _tc_transpose(emt):
    nblk = NMAIN // VB
    main = pl.pallas_call(
        _tc_transpose_body,
        grid=(nblk,),
        in_specs=[pl.BlockSpec((64, VB), lambda i: (0, i))],
        out_specs=pl.BlockSpec((RB, 128), lambda i: (i, 0)),
        out_shape=jax.ShapeDtypeStruct((V // 2, 128), jnp.float32),
    )(emt)
    return pl.pallas_call(
        _tc_tail_body,
        grid=(1,),
        in_specs=[pl.BlockSpec((64, NTAIL), lambda i: (0, 0)),
                  pl.BlockSpec(memory_space=pl.ANY)],
        out_specs=pl.BlockSpec((RB, 128), lambda i: (NMAIN // VB, 0)),
        out_shape=jax.ShapeDtypeStruct((V // 2, 128), jnp.float32),
        input_output_aliases={1: 0},
    )(emt[:, NMAIN:], main)


def _tc_remap_body(i_ref, o_ref):
    v = i_ref[...]
    j = v & (VB - 1)
    main_row = (v - j) + 2 * (j & (RB - 1)) + (j >> 12)
    jt = v - NMAIN
    tail_row = NMAIN + 2 * (jt % HT) + jt // HT
    o_ref[...] = jnp.where(v >= NMAIN, tail_row, main_row)


def _tc_remap(input_ids):
    return pl.pallas_call(
        _tc_remap_body,
        out_shape=jax.ShapeDtypeStruct((B, H), jnp.int32),
    )(input_ids)


def _tc_matmul_body(x_ref, wt_ref, b_ref, o_ref):
    x = x_ref[...] * (1.0 / H)
    o_ref[...] = jnp.dot(x, wt_ref[...],
                         preferred_element_type=jnp.float32) + b_ref[...]


def _tc_matmul(x, wt, b2):
    return pl.pallas_call(
        _tc_matmul_body,
        out_shape=jax.ShapeDtypeStruct((B, NL), jnp.float32),
    )(x, wt, b2)


def kernel(input, emb, W, b):
    lin = _tc_transpose(emb.T).reshape(V, 64)
    pooled = _sc_pool(_tc_remap(input), lin)
    return _tc_matmul(pooled, W.T, b.reshape(1, NL))


# speedup vs baseline: 2.2073x; 1.0855x over previous
import jax
import jax.numpy as jnp
from jax import lax
from jax.experimental import pallas as pl
from jax.experimental.pallas import tpu as pltpu
from jax.experimental.pallas import tpu_sc as plsc

NC, NS = 2, 16
NW = NC * NS
B, H, D, NL = 4096, 200, 64, 128
BPW = B // NW
C0, C1 = 104, 96


def _sc_pool_body(idx_hbm, emb_hbm, out_hbm, idx_v, rows_a, rows_b, acc_v,
                  sem_a, sem_b):
    c = lax.axis_index("c")
    s = lax.axis_index("s")
    wid = s * NC + c
    base = wid * BPW
    pltpu.sync_copy(idx_hbm.at[pl.ds(base, BPW)], idx_v)

    def start(r, rows_ref, sem):
        pltpu.async_copy(
            emb_hbm.at[idx_v.at[r, pl.ds(0, C0)]], rows_ref.at[pl.ds(0, C0)], sem)
        pltpu.async_copy(
            emb_hbm.at[idx_v.at[r, pl.ds(C0, C1)]], rows_ref.at[pl.ds(C0, C1)], sem)

    def wait(r, rows_ref, sem):
        pltpu.make_async_copy(
            emb_hbm.at[idx_v.at[r, pl.ds(0, C0)]], rows_ref.at[pl.ds(0, C0)], sem).wait()
        pltpu.make_async_copy(
            emb_hbm.at[idx_v.at[r, pl.ds(C0, C1)]], rows_ref.at[pl.ds(C0, C1)], sem).wait()

    def reduce(rows_ref, r):
        def jbody(j, accs):
            return tuple(accs[k] + rows_ref[j, pl.ds(16 * k, 16)] for k in range(4))
        accs = lax.fori_loop(
            0, H, jbody,
            tuple(jnp.zeros((16,), jnp.float32) for _ in range(4)),
            unroll=8)
        for k in range(4):
            acc_v[r, pl.ds(16 * k, 16)] = accs[k]

    start(0, rows_a, sem_a)

    def body2(k, carry):
        r = 2 * k
        start(r + 1, rows_b, sem_b)
        wait(r, rows_a, sem_a)
        reduce(rows_a, r)

        @pl.when(k < BPW // 2 - 1)
        def _():
            start(r + 2, rows_a, sem_a)

        wait(r + 1, rows_b, sem_b)
        reduce(rows_b, r + 1)
        return carry

    lax.fori_loop(0, BPW // 2, body2, 0)
    pltpu.sync_copy(acc_v, out_hbm.at[pl.ds(base, BPW)])


def _sc_pool(input_ids, emb):
    mesh = plsc.VectorSubcoreMesh(
        core_axis_name="c", subcore_axis_name="s",
        num_cores=NC, num_subcores=NS)
    return pl.kernel(
        _sc_pool_body,
        out_type=jax.ShapeDtypeStruct((B, D), jnp.float32),
        mesh=mesh,
        scratch_types=[
            pltpu.VMEM((BPW, H), jnp.int32),
            pltpu.VMEM((H, D), jnp.float32),
            pltpu.VMEM((H, D), jnp.float32),
            pltpu.VMEM((BPW, D), jnp.float32),
            pltpu.SemaphoreType.DMA,
            pltpu.SemaphoreType.DMA,
        ],
        compiler_params=pltpu.CompilerParams(use_tc_tiling_on_sc=False),
    )(input_ids, emb)


VB = 16384
RB = VB // 2
V = 1000000
RSH = RB.bit_length() - 1


def _tc_transpose_body(x_ref, o_ref):
    o_ref[:, 0:64] = x_ref[:, 0:RB].T
    o_ref[:, 64:128] = x_ref[:, RB:VB].T


NMAIN = (V // VB) * VB
NTAIL = V - NMAIN
HT = NTAIL // 2


def _tc_tail_body(x_ref, tbl_ref, o_ref):
    del tbl_ref
    o_ref[0:HT, 0:64] = x_ref[:, 0:HT].T
    o_ref[0:HT, 64:128] = x_ref[:, HT:NTAIL].T


def _tc_transpose(emt):
    nblk = NMAIN // VB
    main = pl.pallas_call(
        _tc_transpose_body,
        grid=(nblk,),
        in_specs=[pl.BlockSpec((64, VB), lambda i: (0, i))],
        out_specs=pl.BlockSpec((RB, 128), lambda i: (i, 0)),
        out_shape=jax.ShapeDtypeStruct((V // 2, 128), jnp.float32),
    )(emt)
    return pl.pallas_call(
        _tc_tail_body,
        grid=(1,),
        in_specs=[pl.BlockSpec((64, NTAIL), lambda i: (0, 0)),
                  pl.BlockSpec(memory_space=pl.ANY)],
        out_specs=pl.BlockSpec((RB, 128), lambda i: (NMAIN // VB, 0)),
        out_shape=jax.ShapeDtypeStruct((V // 2, 128), jnp.float32),
        input_output_aliases={1: 0},
    )(emt[:, NMAIN:], main)


def _tc_remap_body(i_ref, o_ref):
    v = i_ref[...]
    j = v & (VB - 1)
    main_row = (v - j) + 2 * (j & (RB - 1)) + (j >> RSH)
    jt = v - NMAIN
    tail_row = NMAIN + 2 * (jt % HT) + jt // HT
    o_ref[...] = jnp.where(v >= NMAIN, tail_row, main_row)


def _tc_remap(input_ids):
    return pl.pallas_call(
        _tc_remap_body,
        out_shape=jax.ShapeDtypeStruct((B, H), jnp.int32),
    )(input_ids)


def _tc_matmul_body(x_ref, wt_ref, b_ref, o_ref):
    x = x_ref[...] * (1.0 / H)
    o_ref[...] = jnp.dot(x, wt_ref[...],
                         preferred_element_type=jnp.float32) + b_ref[...]


def _tc_matmul(x, wt, b2):
    return pl.pallas_call(
        _tc_matmul_body,
        out_shape=jax.ShapeDtypeStruct((B, NL), jnp.float32),
    )(x, wt, b2)


def kernel(input, emb, W, b):
    lin = _tc_transpose(emb.T).reshape(V, 64)
    pooled = _sc_pool(_tc_remap(input), lin)
    return _tc_matmul(pooled, W.T, b.reshape(1, NL))


# speedup vs baseline: 2.2626x; 1.0250x over previous
import jax
import jax.numpy as jnp
from jax import lax
from jax.experimental import pallas as pl
from jax.experimental.pallas import tpu as pltpu
from jax.experimental.pallas import tpu_sc as plsc

NC, NS = 2, 16
NW = NC * NS
B, H, D, NL = 4096, 200, 64, 128
BPW = B // NW
C0, C1 = 104, 96


def _sc_pool_body(idx_hbm, emb_hbm, out_hbm, idx_v, rows_a, rows_b, acc_v,
                  sem_a, sem_b):
    c = lax.axis_index("c")
    s = lax.axis_index("s")
    wid = s * NC + c
    base = wid * BPW
    pltpu.sync_copy(idx_hbm.at[pl.ds(base, BPW)], idx_v)

    def start(r, rows_ref, sem):
        pltpu.async_copy(
            emb_hbm.at[idx_v.at[r, pl.ds(0, C0)]], rows_ref.at[pl.ds(0, C0)], sem)
        pltpu.async_copy(
            emb_hbm.at[idx_v.at[r, pl.ds(C0, C1)]], rows_ref.at[pl.ds(C0, C1)], sem)

    def wait(r, rows_ref, sem):
        pltpu.make_async_copy(
            emb_hbm.at[idx_v.at[r, pl.ds(0, C0)]], rows_ref.at[pl.ds(0, C0)], sem).wait()
        pltpu.make_async_copy(
            emb_hbm.at[idx_v.at[r, pl.ds(C0, C1)]], rows_ref.at[pl.ds(C0, C1)], sem).wait()

    def reduce(rows_ref, r):
        def jbody(j, accs):
            return tuple(accs[k] + rows_ref[j, pl.ds(16 * k, 16)] for k in range(4))
        accs = lax.fori_loop(
            0, H, jbody,
            tuple(jnp.zeros((16,), jnp.float32) for _ in range(4)),
            unroll=8)
        for k in range(4):
            acc_v[r, pl.ds(16 * k, 16)] = accs[k]

    start(0, rows_a, sem_a)

    def body2(k, carry):
        r = 2 * k
        start(r + 1, rows_b, sem_b)
        wait(r, rows_a, sem_a)
        reduce(rows_a, r)

        @pl.when(k < BPW // 2 - 1)
        def _():
            start(r + 2, rows_a, sem_a)

        wait(r + 1, rows_b, sem_b)
        reduce(rows_b, r + 1)
        return carry

    lax.fori_loop(0, BPW // 2, body2, 0)
    pltpu.sync_copy(acc_v, out_hbm.at[pl.ds(base, BPW)])


def _sc_pool(input_ids, emb):
    mesh = plsc.VectorSubcoreMesh(
        core_axis_name="c", subcore_axis_name="s",
        num_cores=NC, num_subcores=NS)
    return pl.kernel(
        _sc_pool_body,
        out_type=jax.ShapeDtypeStruct((B, D), jnp.float32),
        mesh=mesh,
        scratch_types=[
            pltpu.VMEM((BPW, H), jnp.int32),
            pltpu.VMEM((H, D), jnp.float32),
            pltpu.VMEM((H, D), jnp.float32),
            pltpu.VMEM((BPW, D), jnp.float32),
            pltpu.SemaphoreType.DMA,
            pltpu.SemaphoreType.DMA,
        ],
        compiler_params=pltpu.CompilerParams(use_tc_tiling_on_sc=False),
    )(input_ids, emb)


VB = 32768
RB = VB // 2
V = 1000000
RSH = RB.bit_length() - 1


def _tc_transpose_body(x_ref, o_ref):
    o_ref[:, 0:64] = x_ref[:, 0:RB].T
    o_ref[:, 64:128] = x_ref[:, RB:VB].T


NMAIN = (V // VB) * VB
NTAIL = V - NMAIN
HT = NTAIL // 2


def _tc_tail_body(x_ref, tbl_ref, o_ref):
    del tbl_ref
    o_ref[0:HT, 0:64] = x_ref[:, 0:HT].T
    o_ref[0:HT, 64:128] = x_ref[:, HT:NTAIL].T


def _tc_transpose(emt):
    nblk = NMAIN // VB
    main = pl.pallas_call(
        _tc_transpose_body,
        grid=(nblk,),
        in_specs=[pl.BlockSpec((64, VB), lambda i: (0, i))],
        out_specs=pl.BlockSpec((RB, 128), lambda i: (i, 0)),
        out_shape=jax.ShapeDtypeStruct((V // 2, 128), jnp.float32),
    )(emt)
    return pl.pallas_call(
        _tc_tail_body,
        grid=(1,),
        in_specs=[pl.BlockSpec((64, NTAIL), lambda i: (0, 0)),
                  pl.BlockSpec(memory_space=pl.ANY)],
        out_specs=pl.BlockSpec((RB, 128), lambda i: (NMAIN // VB, 0)),
        out_shape=jax.ShapeDtypeStruct((V // 2, 128), jnp.float32),
        input_output_aliases={1: 0},
    )(emt[:, NMAIN:], main)


def _tc_remap_body(i_ref, o_ref):
    v = i_ref[...]
    j = v & (VB - 1)
    main_row = (v - j) + 2 * (j & (RB - 1)) + (j >> RSH)
    jt = v - NMAIN
    tail_row = NMAIN + 2 * (jt % HT) + jt // HT
    o_ref[...] = jnp.where(v >= NMAIN, tail_row, main_row)


def _tc_remap(input_ids):
    return pl.pallas_call(
        _tc_remap_body,
        out_shape=jax.ShapeDtypeStruct((B, H), jnp.int32),
    )(input_ids)


def _tc_matmul_body(x_ref, wt_ref, b_ref, o_ref):
    x = x_ref[...] * (1.0 / H)
    o_ref[...] = jnp.dot(x, wt_ref[...],
                         preferred_element_type=jnp.float32) + b_ref[...]


def _tc_matmul(x, wt, b2):
    return pl.pallas_call(
        _tc_matmul_body,
        out_shape=jax.ShapeDtypeStruct((B, NL), jnp.float32),
    )(x, wt, b2)


def kernel(input, emb, W, b):
    lin = _tc_transpose(emb.T).reshape(V, 64)
    pooled = _sc_pool(_tc_remap(input), lin)
    return _tc_matmul(pooled, W.T, b.reshape(1, NL))
